# baseline (speedup 1.0000x reference)
import functools

import jax
import jax.numpy as jnp
from jax import lax
from jax.experimental import pallas as pl
from jax.experimental.pallas import tpu as pltpu
from jax.experimental.pallas import tpu_sc as plsc

N = 10000
D = 128
E = 320000
G = 64

NC = 2
NS = 16
NW = NC * NS
EPW = E // NW
CH = 40
NCH = EPW // CH
NPAD = 10240
RPS = NPAD // NS
ZFULL = RPS // CH
ZREM = RPS - ZFULL * CH


def _sc_aggregate(x, src3, dst3):
    mesh = plsc.VectorSubcoreMesh(core_axis_name="c", subcore_axis_name="s")

    @functools.partial(
        pl.kernel,
        out_type=jax.ShapeDtypeStruct((NC, NPAD, D), jnp.float32),
        mesh=mesh,
        compiler_params=pltpu.CompilerParams(use_tc_tiling_on_sc=False),
        scratch_types=[
            pltpu.VMEM((NCH, CH), jnp.int32),
            pltpu.VMEM((NCH, CH), jnp.int32),
            pltpu.VMEM((CH, D), jnp.float32),
            pltpu.VMEM((CH, D), jnp.float32),
            pltpu.VMEM_SHARED((NPAD, D), jnp.float32),
            pltpu.SemaphoreType.DMA,
            pltpu.SemaphoreType.DMA,
        ],
    )
    def agg_kernel(x_hbm, src_hbm, dst_hbm, out_hbm,
                   src_v, dst_v, bufa, bufb, agg_sh, sema, semb):
        c = lax.axis_index("c")
        s = lax.axis_index("s")
        wid = s * NC + c
        base = s * RPS

        @pl.loop(0, CH)
        def _(r):
            @pl.loop(0, D // 16)
            def _(k):
                bufa[r, pl.ds(k * 16, 16)] = jnp.zeros((16,), jnp.float32)

        @pl.loop(0, ZFULL)
        def _(z):
            pltpu.sync_copy(bufa, agg_sh.at[pl.ds(base + z * CH, CH)])

        pltpu.sync_copy(src_hbm.at[wid], src_v)
        pltpu.sync_copy(dst_hbm.at[wid], dst_v)

        plsc.subcore_barrier()

        pltpu.async_copy(x_hbm.at[src_v.at[0]], bufa, sema)

        @pl.loop(0, NCH - 2, step=2)
        def _(j):
            pltpu.make_async_copy(x_hbm.at[src_v.at[0]], bufa, sema).wait()
            pltpu.async_copy(x_hbm.at[src_v.at[j + 1]], bufb, semb)
            pltpu.sync_copy(bufa, agg_sh.at[dst_v.at[j]], add=True)
            pltpu.make_async_copy(x_hbm.at[src_v.at[0]], bufb, semb).wait()
            pltpu.async_copy(x_hbm.at[src_v.at[j + 2]], bufa, sema)
            pltpu.sync_copy(bufb, agg_sh.at[dst_v.at[j + 1]], add=True)

        pltpu.make_async_copy(x_hbm.at[src_v.at[0]], bufa, sema).wait()
        pltpu.async_copy(x_hbm.at[src_v.at[NCH - 1]], bufb, semb)
        pltpu.sync_copy(bufa, agg_sh.at[dst_v.at[NCH - 2]], add=True)
        pltpu.make_async_copy(x_hbm.at[src_v.at[0]], bufb, semb).wait()
        pltpu.sync_copy(bufb, agg_sh.at[dst_v.at[NCH - 1]], add=True)

        plsc.subcore_barrier()

        pltpu.sync_copy(agg_sh.at[pl.ds(base, RPS)],
                        out_hbm.at[c].at[pl.ds(base, RPS)])

    return agg_kernel(x, src3, dst3)


def _tc_head(x, agg2, batch2, W1, b1r, w2r, b2r):
    def body(x_ref, agg_ref, batch_ref, w1_ref, b1_ref, w2_ref, b2_ref,
             out_ref):
        a = x_ref[...] + agg_ref[0, :N, :] + agg_ref[1, :N, :]
        h = jnp.dot(a, w1_ref[...], preferred_element_type=jnp.float32)
        h = jnp.maximum(h + b1_ref[...], 0.0)
        y = jnp.sum(h * w2_ref[...], axis=1, keepdims=True)
        gids = lax.broadcasted_iota(jnp.int32, (1, G), 1)
        contrib = jnp.where(batch_ref[...] == gids, y, 0.0)
        pooled = jnp.sum(contrib, axis=0)
        out_ref[...] = pooled[:, None] + b2_ref[0, 0]

    return pl.pallas_call(
        body,
        out_shape=jax.ShapeDtypeStruct((G, 1), jnp.float32),
    )(x, agg2, batch2, W1, b1r, w2r, b2r)


def kernel(x, edge_index, batch, W1, b1, W2, b2):
    src3 = edge_index[0].reshape(NW, NCH, CH)
    dst3 = edge_index[1].reshape(NW, NCH, CH)
    agg2 = _sc_aggregate(x, src3, dst3)
    return _tc_head(x, agg2, batch.reshape(N, 1), W1,
                    b1.reshape(1, D), W2.reshape(1, D), b2.reshape(1, 1))


# speedup vs baseline: 1.4892x; 1.4892x over previous
import functools

import jax
import jax.numpy as jnp
from jax import lax
from jax.experimental import pallas as pl
from jax.experimental.pallas import tpu as pltpu
from jax.experimental.pallas import tpu_sc as plsc

N = 10000
D = 128
E = 320000
G = 64

NC = 2
NS = 16
DH = D // NC
EPW = E // NS
CH = 125
NCH = EPW // CH
NB = 4
NPAD = 10240
RPS = NPAD // NS
ZFULL = RPS // CH
ZREM = RPS - ZFULL * CH


def _sc_aggregate(xh, src3, dst3):
    mesh = plsc.VectorSubcoreMesh(core_axis_name="c", subcore_axis_name="s")

    @functools.partial(
        pl.kernel,
        out_type=jax.ShapeDtypeStruct((NC, NPAD, DH), jnp.float32),
        mesh=mesh,
        compiler_params=pltpu.CompilerParams(use_tc_tiling_on_sc=False),
        scratch_types=[
            pltpu.VMEM((NCH, CH), jnp.int32),
            pltpu.VMEM((NCH, CH), jnp.int32),
            pltpu.VMEM((NB, CH, DH), jnp.float32),
            pltpu.VMEM_SHARED((NPAD, DH), jnp.float32),
            pltpu.SemaphoreType.DMA((NB,)),
            pltpu.SemaphoreType.DMA((NB,)),
        ],
    )
    def agg_kernel(x_hbm, src_hbm, dst_hbm, out_hbm,
                   src_v, dst_v, bufs, agg_sh, gsem, ssem):
        c = lax.axis_index("c")
        s = lax.axis_index("s")
        base = s * RPS

        @pl.loop(0, CH)
        def _(r):
            @pl.loop(0, DH // 16)
            def _(k):
                bufs[0, r, pl.ds(k * 16, 16)] = jnp.zeros((16,), jnp.float32)

        @pl.loop(0, ZFULL)
        def _(z):
            pltpu.sync_copy(bufs.at[0], agg_sh.at[pl.ds(base + z * CH, CH)])

        pltpu.sync_copy(bufs.at[0, pl.ds(0, ZREM)],
                        agg_sh.at[pl.ds(base + ZFULL * CH, ZREM)])

        pltpu.sync_copy(src_hbm.at[s], src_v)
        pltpu.sync_copy(dst_hbm.at[s], dst_v)

        plsc.subcore_barrier()

        xc = x_hbm.at[c]

        def fire_gather(b, j):
            pltpu.async_copy(xc.at[src_v.at[j]], bufs.at[b], gsem.at[b])

        def wait_gather(b):
            pltpu.make_async_copy(xc.at[src_v.at[0]], bufs.at[b],
                                  gsem.at[b]).wait()

        def fire_scatter(b, j):
            pltpu.async_copy(bufs.at[b], agg_sh.at[dst_v.at[j]],
                             ssem.at[b], add=True)

        def wait_scatter(b):
            pltpu.make_async_copy(bufs.at[b], agg_sh.at[dst_v.at[0]],
                                  ssem.at[b]).wait()

        for b in range(NB):
            fire_gather(b, b)

        @pl.loop(0, NCH - NB, step=NB)
        def _(j):
            for b in range(NB):
                wait_gather(b)
                fire_scatter(b, j + b)
            for b in range(NB):
                wait_scatter(b)
                fire_gather(b, j + NB + b)

        for b in range(NB):
            wait_gather(b)
            fire_scatter(b, NCH - NB + b)
        for b in range(NB):
            wait_scatter(b)

        plsc.subcore_barrier()

        pltpu.sync_copy(agg_sh.at[pl.ds(base, RPS)],
                        out_hbm.at[c].at[pl.ds(base, RPS)])

    return agg_kernel(xh, src3, dst3)


def _tc_head(x, agg2, batch2, W1, b1r, w2r, b2r):
    def body(x_ref, agg_ref, batch_ref, w1_ref, b1_ref, w2_ref, b2_ref,
             out_ref):
        w1 = w1_ref[...]
        h = jnp.dot(x_ref[...], w1, preferred_element_type=jnp.float32)
        h += jnp.dot(agg_ref[0, :N, :], w1[:DH, :],
                     preferred_element_type=jnp.float32)
        h += jnp.dot(agg_ref[1, :N, :], w1[DH:, :],
                     preferred_element_type=jnp.float32)
        h = jnp.maximum(h + b1_ref[...], 0.0)
        y = jnp.sum(h * w2_ref[...], axis=1, keepdims=True)
        gids = lax.broadcasted_iota(jnp.int32, (1, G), 1)
        contrib = jnp.where(batch_ref[...] == gids, y, 0.0)
        pooled = jnp.sum(contrib, axis=0)
        out_ref[...] = pooled[:, None] + b2_ref[0, 0]

    return pl.pallas_call(
        body,
        out_shape=jax.ShapeDtypeStruct((G, 1), jnp.float32),
    )(x, agg2, batch2, W1, b1r, w2r, b2r)


def kernel(x, edge_index, batch, W1, b1, W2, b2):
    xh = jnp.stack([x[:, :DH], x[:, DH:]])
    src3 = edge_index[0].reshape(NS, NCH, CH)
    dst3 = edge_index[1].reshape(NS, NCH, CH)
    agg2 = _sc_aggregate(xh, src3, dst3)
    return _tc_head(x, agg2, batch.reshape(N, 1), W1,
                    b1.reshape(1, D), W2.reshape(1, D), b2.reshape(1, 1))


# speedup vs baseline: 1.5100x; 1.0139x over previous
import functools

import jax
import jax.numpy as jnp
from jax import lax
from jax.experimental import pallas as pl
from jax.experimental.pallas import tpu as pltpu
from jax.experimental.pallas import tpu_sc as plsc

N = 10000
D = 128
E = 320000
G = 64

NC = 2
NS = 16
DH = D // NC
EPW = E // NS
CH = 125
NCH = EPW // CH
NB = 5
NPAD = 10240
RPS = NPAD // NS
ZFULL = RPS // CH
ZREM = RPS - ZFULL * CH


def _sc_aggregate(xh, src3, dst3):
    mesh = plsc.VectorSubcoreMesh(core_axis_name="c", subcore_axis_name="s")

    @functools.partial(
        pl.kernel,
        out_type=jax.ShapeDtypeStruct((NC, NPAD, DH), jnp.float32),
        mesh=mesh,
        compiler_params=pltpu.CompilerParams(use_tc_tiling_on_sc=False),
        scratch_types=[
            pltpu.VMEM((NCH, CH), jnp.int32),
            pltpu.VMEM((NCH, CH), jnp.int32),
            pltpu.VMEM((NB, CH, DH), jnp.float32),
            pltpu.VMEM_SHARED((NPAD, DH), jnp.float32),
            pltpu.SemaphoreType.DMA((NB,)),
            pltpu.SemaphoreType.DMA((NB,)),
        ],
    )
    def agg_kernel(x_hbm, src_hbm, dst_hbm, out_hbm,
                   src_v, dst_v, bufs, agg_sh, gsem, ssem):
        c = lax.axis_index("c")
        s = lax.axis_index("s")
        base = s * RPS

        @pl.loop(0, CH)
        def _(r):
            @pl.loop(0, DH // 16)
            def _(k):
                bufs[0, r, pl.ds(k * 16, 16)] = jnp.zeros((16,), jnp.float32)

        @pl.loop(0, ZFULL)
        def _(z):
            pltpu.sync_copy(bufs.at[0], agg_sh.at[pl.ds(base + z * CH, CH)])

        pltpu.sync_copy(bufs.at[0, pl.ds(0, ZREM)],
                        agg_sh.at[pl.ds(base + ZFULL * CH, ZREM)])

        pltpu.sync_copy(src_hbm.at[s], src_v)
        pltpu.sync_copy(dst_hbm.at[s], dst_v)

        plsc.subcore_barrier()

        xc = x_hbm.at[c]

        def fire_gather(b, j):
            pltpu.async_copy(xc.at[src_v.at[j]], bufs.at[b], gsem.at[b])

        def wait_gather(b):
            pltpu.make_async_copy(xc.at[src_v.at[0]], bufs.at[b],
                                  gsem.at[b]).wait()

        def fire_scatter(b, j):
            pltpu.async_copy(bufs.at[b], agg_sh.at[dst_v.at[j]],
                             ssem.at[b], add=True)

        def wait_scatter(b):
            pltpu.make_async_copy(bufs.at[b], agg_sh.at[dst_v.at[0]],
                                  ssem.at[b]).wait()

        for b in range(NB):
            fire_gather(b, b)

        @pl.loop(0, NCH - NB, step=NB)
        def _(j):
            for b in range(NB):
                wait_gather(b)
                fire_scatter(b, j + b)
            for b in range(NB):
                wait_scatter(b)
                fire_gather(b, j + NB + b)

        for b in range(NB):
            wait_gather(b)
            fire_scatter(b, NCH - NB + b)
        for b in range(NB):
            wait_scatter(b)

        plsc.subcore_barrier()

        pltpu.sync_copy(agg_sh.at[pl.ds(base, RPS)],
                        out_hbm.at[c].at[pl.ds(base, RPS)])

    return agg_kernel(xh, src3, dst3)


def _tc_head(x, agg2, batch2, W1, b1r, w2r, b2r):
    def body(x_ref, agg_ref, batch_ref, w1_ref, b1_ref, w2_ref, b2_ref,
             out_ref):
        w1 = w1_ref[...]
        h = jnp.dot(x_ref[...], w1, preferred_element_type=jnp.float32)
        h += jnp.dot(agg_ref[0, :N, :], w1[:DH, :],
                     preferred_element_type=jnp.float32)
        h += jnp.dot(agg_ref[1, :N, :], w1[DH:, :],
                     preferred_element_type=jnp.float32)
        h = jnp.maximum(h + b1_ref[...], 0.0)
        y = jnp.sum(h * w2_ref[...], axis=1, keepdims=True)
        gids = lax.broadcasted_iota(jnp.int32, (1, G), 1)
        contrib = jnp.where(batch_ref[...] == gids, y, 0.0)
        pooled = jnp.sum(contrib, axis=0)
        out_ref[...] = pooled[:, None] + b2_ref[0, 0]

    return pl.pallas_call(
        body,
        out_shape=jax.ShapeDtypeStruct((G, 1), jnp.float32),
    )(x, agg2, batch2, W1, b1r, w2r, b2r)


def kernel(x, edge_index, batch, W1, b1, W2, b2):
    xh = jnp.stack([x[:, :DH], x[:, DH:]])
    src3 = edge_index[0].reshape(NS, NCH, CH)
    dst3 = edge_index[1].reshape(NS, NCH, CH)
    agg2 = _sc_aggregate(xh, src3, dst3)
    return _tc_head(x, agg2, batch.reshape(N, 1), W1,
                    b1.reshape(1, D), W2.reshape(1, D), b2.reshape(1, 1))


# speedup vs baseline: 1.6444x; 1.0890x over previous
import functools

import jax
import jax.numpy as jnp
from jax import lax
from jax.experimental import pallas as pl
from jax.experimental.pallas import tpu as pltpu
from jax.experimental.pallas import tpu_sc as plsc

N = 10000
D = 128
E = 320000
G = 64

NC = 2
NS = 16
DH = D // NC
EPW = E // NS
CH = 80
NCH = EPW // CH
NB = 5
NPAD = 10240
RPS = NPAD // NS
ZFULL = RPS // CH
ZREM = RPS - ZFULL * CH


def _sc_aggregate(x2, srcs, dsts):
    mesh = plsc.VectorSubcoreMesh(core_axis_name="c", subcore_axis_name="s")

    @functools.partial(
        pl.kernel,
        out_type=jax.ShapeDtypeStruct((NC, NPAD, DH), jnp.float32),
        mesh=mesh,
        compiler_params=pltpu.CompilerParams(use_tc_tiling_on_sc=False),
        scratch_types=[
            pltpu.VMEM((EPW,), jnp.int32),
            pltpu.VMEM((NCH, CH), jnp.int32),
            pltpu.VMEM((NB, CH, DH), jnp.float32),
            pltpu.VMEM_SHARED((NPAD, DH), jnp.float32),
            pltpu.SemaphoreType.DMA((NB,)),
            pltpu.SemaphoreType.DMA((NB,)),
        ],
    )
    def agg_kernel(x_hbm, src_hbm, dst_hbm, out_hbm,
                   src_v, dst_v, bufs, agg_sh, gsem, ssem):
        c = lax.axis_index("c")
        s = lax.axis_index("s")
        base = s * RPS

        pltpu.sync_copy(src_hbm.at[s], src_v)
        pltpu.sync_copy(dst_hbm.at[s], dst_v)

        @pl.loop(0, EPW // 16)
        def _(t):
            sl = pl.ds(t * 16, 16)
            src_v[sl] = src_v[sl] * 2 + c

        @pl.loop(0, CH)
        def _(r):
            @pl.loop(0, DH // 16)
            def _(k):
                bufs[0, r, pl.ds(k * 16, 16)] = jnp.zeros((16,), jnp.float32)

        @pl.loop(0, ZFULL)
        def _(z):
            pltpu.sync_copy(bufs.at[0], agg_sh.at[pl.ds(base + z * CH, CH)])

        if ZREM:
            pltpu.sync_copy(bufs.at[0, pl.ds(0, ZREM)],
                            agg_sh.at[pl.ds(base + ZFULL * CH, ZREM)])

        plsc.subcore_barrier()

        def fire_gather(b, j):
            pltpu.async_copy(x_hbm.at[src_v.at[pl.ds(j * CH, CH)]],
                             bufs.at[b], gsem.at[b])

        def wait_gather(b):
            pltpu.make_async_copy(x_hbm.at[src_v.at[pl.ds(0, CH)]],
                                  bufs.at[b], gsem.at[b]).wait()

        def fire_scatter(b, j):
            pltpu.async_copy(bufs.at[b], agg_sh.at[dst_v.at[j]],
                             ssem.at[b], add=True)

        def wait_scatter(b):
            pltpu.make_async_copy(bufs.at[b], agg_sh.at[dst_v.at[0]],
                                  ssem.at[b]).wait()

        for b in range(NB):
            fire_gather(b, b)

        @pl.loop(0, NCH - NB, step=NB)
        def _(j):
            for b in range(NB):
                wait_gather(b)
                fire_scatter(b, j + b)
            for b in range(NB):
                wait_scatter(b)
                fire_gather(b, j + NB + b)

        for b in range(NB):
            wait_gather(b)
            fire_scatter(b, NCH - NB + b)
        for b in range(NB):
            wait_scatter(b)

        plsc.subcore_barrier()

        pltpu.sync_copy(agg_sh.at[pl.ds(base, RPS)],
                        out_hbm.at[c].at[pl.ds(base, RPS)])

    return agg_kernel(x2, srcs, dsts)


def _tc_head(x, agg2, batch2, W1, b1r, w2r, b2r):
    def body(x_ref, agg_ref, batch_ref, w1_ref, b1_ref, w2_ref, b2_ref,
             out_ref):
        w1 = w1_ref[...]
        h = jnp.dot(x_ref[...], w1, preferred_element_type=jnp.float32)
        h += jnp.dot(agg_ref[0, :N, :], w1[:DH, :],
                     preferred_element_type=jnp.float32)
        h += jnp.dot(agg_ref[1, :N, :], w1[DH:, :],
                     preferred_element_type=jnp.float32)
        h = jnp.maximum(h + b1_ref[...], 0.0)
        y = jnp.sum(h * w2_ref[...], axis=1, keepdims=True)
        gids = lax.broadcasted_iota(jnp.int32, (1, G), 1)
        contrib = jnp.where(batch_ref[...] == gids, y, 0.0)
        pooled = jnp.sum(contrib, axis=0)
        out_ref[...] = pooled[:, None] + b2_ref[0, 0]

    return pl.pallas_call(
        body,
        out_shape=jax.ShapeDtypeStruct((G, 1), jnp.float32),
    )(x, agg2, batch2, W1, b1r, w2r, b2r)


def kernel(x, edge_index, batch, W1, b1, W2, b2):
    x2 = x.reshape(2 * N, DH)
    srcs = edge_index[0].reshape(NS, EPW)
    dsts = edge_index[1].reshape(NS, NCH, CH)
    agg2 = _sc_aggregate(x2, srcs, dsts)
    return _tc_head(x, agg2, batch.reshape(N, 1), W1,
                    b1.reshape(1, D), W2.reshape(1, D), b2.reshape(1, 1))
